# baseline (speedup 1.0000x reference)
import functools

import jax
import jax.numpy as jnp
from jax.experimental import pallas as pl
from jax.experimental.pallas import tpu as pltpu

N = 4096
D = 256
R = 3
ROW_TILE = 64


def _fused_body(feature_ref, adj_ref, w_ref, out_ref, h_ref):
    i = pl.program_id(0)

    @pl.when(i == 0)
    def _compute_h():
        h_ref[...] = jnp.tanh(
            jnp.dot(feature_ref[...], w_ref[...],
                    preferred_element_type=jnp.float32))

    a = adj_ref[0] + adj_ref[1] + adj_ref[2]
    out_ref[...] = jnp.dot(a, h_ref[...], preferred_element_type=jnp.float32)


@jax.jit
def kernel(feature, same_type_adj, W, b):
    del b
    grid = (N // ROW_TILE,)
    return pl.pallas_call(
        _fused_body,
        grid=grid,
        in_specs=[
            pl.BlockSpec((N, D), lambda i: (0, 0)),
            pl.BlockSpec((R, ROW_TILE, N), lambda i: (0, i, 0)),
            pl.BlockSpec((D, D), lambda i: (0, 0)),
        ],
        out_specs=pl.BlockSpec((ROW_TILE, D), lambda i: (i, 0)),
        out_shape=jax.ShapeDtypeStruct((N, D), jnp.float32),
        scratch_shapes=[pltpu.VMEM((N, D), jnp.float32)],
    )(feature, same_type_adj, W)


# speedup vs baseline: 1.2961x; 1.2961x over previous
import functools

import jax
import jax.numpy as jnp
from jax.experimental import pallas as pl
from jax.experimental.pallas import tpu as pltpu

N = 4096
D = 256
R = 3
ROW_TILE = 128


def _fused_body(feature_ref, a0_ref, a1_ref, a2_ref, w_ref, out_ref, h_ref):
    i = pl.program_id(0)

    @pl.when(i == 0)
    def _compute_h():
        h_ref[...] = jnp.tanh(
            jnp.dot(feature_ref[...], w_ref[...],
                    preferred_element_type=jnp.float32))

    a = a0_ref[0] + a1_ref[0] + a2_ref[0]
    out_ref[...] = jnp.dot(a, h_ref[...], preferred_element_type=jnp.float32)


@jax.jit
def kernel(feature, same_type_adj, W, b):
    del b
    grid = (N // ROW_TILE,)
    adj_spec = lambda r: pl.BlockSpec((1, ROW_TILE, N), lambda i, r=r: (r, i, 0))
    return pl.pallas_call(
        _fused_body,
        grid=grid,
        in_specs=[
            pl.BlockSpec((N, D), lambda i: (0, 0)),
            adj_spec(0), adj_spec(1), adj_spec(2),
            pl.BlockSpec((D, D), lambda i: (0, 0)),
        ],
        out_specs=pl.BlockSpec((ROW_TILE, D), lambda i: (i, 0)),
        out_shape=jax.ShapeDtypeStruct((N, D), jnp.float32),
        scratch_shapes=[pltpu.VMEM((N, D), jnp.float32)],
    )(feature, same_type_adj, same_type_adj, same_type_adj, W)


# speedup vs baseline: 1.3057x; 1.0074x over previous
import functools

import jax
import jax.numpy as jnp
from jax.experimental import pallas as pl
from jax.experimental.pallas import tpu as pltpu

N = 4096
D = 256
R = 3
ROW_TILE = 128


def _fused_body(feature_ref, adj_ref, w_ref, out_ref, h_ref):
    i = pl.program_id(0)

    @pl.when(i == 0)
    def _compute_h():
        h_ref[...] = jnp.tanh(
            jnp.dot(feature_ref[...], w_ref[...],
                    preferred_element_type=jnp.float32))

    a = adj_ref[0] + adj_ref[1] + adj_ref[2]
    out_ref[...] = jnp.dot(a, h_ref[...], preferred_element_type=jnp.float32)


@jax.jit
def kernel(feature, same_type_adj, W, b):
    del b
    grid = (N // ROW_TILE,)
    return pl.pallas_call(
        _fused_body,
        grid=grid,
        in_specs=[
            pl.BlockSpec((N, D), lambda i: (0, 0)),
            pl.BlockSpec((R, ROW_TILE, N), lambda i: (0, i, 0)),
            pl.BlockSpec((D, D), lambda i: (0, 0)),
        ],
        out_specs=pl.BlockSpec((ROW_TILE, D), lambda i: (i, 0)),
        out_shape=jax.ShapeDtypeStruct((N, D), jnp.float32),
        scratch_shapes=[pltpu.VMEM((N, D), jnp.float32)],
    )(feature, same_type_adj, W)
